# Initial kernel scaffold; baseline (speedup 1.0000x reference)
import jax
import jax.numpy as jnp
from jax.experimental import pallas as pl


def kernel(inputs, adj_mat, W, w1, w2, b):
    raise NotImplementedError("write your pallas kernel here")


# speedup vs baseline: 2.1211x; 2.1211x over previous
import jax
import jax.numpy as jnp
from jax.experimental import pallas as pl
from jax.experimental.pallas import tpu as pltpu

_N = 10000
_D = 128
_BR = 400
_BC = 2048
_NRB = _N // _BR
_NCB = -(-_N // _BC)
_BP = 2000


def _proj_kernel(in_ref, w_ref, wf_ref, x_ref, f_ref):
    x = jnp.dot(in_ref[...], w_ref[...], preferred_element_type=jnp.float32)
    x_ref[...] = x
    ff = jnp.dot(x, wf_ref[...], preferred_element_type=jnp.float32)
    f_ref[...] = jnp.where(ff > 0, ff, jnp.exp(ff) - 1.0)


def _gat_kernel(adj_ref, xv_ref, f1m_ref, f2t_ref, b_ref, out_ref,
                acc_ref, ssum_ref):
    j = pl.program_id(1)

    @pl.when(j == 0)
    def _():
        acc_ref[...] = jnp.zeros_like(acc_ref)
        ssum_ref[...] = jnp.zeros_like(ssum_ref)

    adj = adj_ref[...]
    col = jax.lax.broadcasted_iota(jnp.int32, (_BR, _BC), 1)
    valid = col < (_N - j * _BC)
    neg_inf = jnp.float32(-jnp.inf)
    xv = xv_ref[...]
    for h in range(2):
        srow = f1m_ref[:, h:h + 1]
        m = f1m_ref[:, 2 + h:3 + h]
        scol = f2t_ref[h:h + 1, :]
        z = adj * (srow + scol)
        z = jnp.maximum(z, 0.2 * z)
        z = jnp.where(valid, z - m, neg_inf)
        e = jnp.exp(z)
        ssum_ref[:, h:h + 1] += jnp.sum(e, axis=1, keepdims=True)
        acc_ref[h, :, :] += jnp.dot(e, xv, preferred_element_type=jnp.float32)

    @pl.when(j == _NCB - 1)
    def _():
        res = None
        for h in range(2):
            v = acc_ref[h, :, :] / ssum_ref[:, h:h + 1] + b_ref[h:h + 1, :]
            v = jnp.where(v > 0, v, jnp.exp(v) - 1.0)
            res = v if res is None else res + v
        out_ref[...] = res * 0.5


def kernel(inputs, adj_mat, W, w1, w2, b):
    wf = jnp.concatenate(
        [w1[0], w1[1], w2[0], w2[1], jnp.zeros((_D, 4), jnp.float32)], axis=1)
    x, f = pl.pallas_call(
        _proj_kernel,
        grid=(_N // _BP,),
        in_specs=[pl.BlockSpec((_BP, _D), lambda i: (i, 0)),
                  pl.BlockSpec((_D, _D), lambda i: (0, 0)),
                  pl.BlockSpec((_D, 8), lambda i: (0, 0))],
        out_specs=[pl.BlockSpec((_BP, _D), lambda i: (i, 0)),
                   pl.BlockSpec((_BP, 8), lambda i: (i, 0))],
        out_shape=(jax.ShapeDtypeStruct((_N, _D), jnp.float32),
                   jax.ShapeDtypeStruct((_N, 8), jnp.float32)),
    )(inputs, W, wf)

    f1 = f[:, 0:2]
    f2 = f[:, 2:4]
    m = jnp.maximum(f1 + jnp.max(f2, axis=0)[None, :], 0.0)
    f1m = jnp.concatenate([f1, m], axis=1)
    npad = _NCB * _BC - _N
    x_pad = jnp.pad(x, ((0, npad), (0, 0)))
    f2t = jnp.pad(f2.T, ((0, 6), (0, npad)))

    out = pl.pallas_call(
        _gat_kernel,
        grid=(_NRB, _NCB),
        in_specs=[pl.BlockSpec((_BR, _BC), lambda i, j: (i, j)),
                  pl.BlockSpec((_BC, _D), lambda i, j: (j, 0)),
                  pl.BlockSpec((_BR, 4), lambda i, j: (i, 0)),
                  pl.BlockSpec((8, _BC), lambda i, j: (0, j)),
                  pl.BlockSpec((2, _D), lambda i, j: (0, 0))],
        out_specs=pl.BlockSpec((_BR, _D), lambda i, j: (i, 0)),
        out_shape=jax.ShapeDtypeStruct((_N, _D), jnp.float32),
        scratch_shapes=[pltpu.VMEM((2, _BR, _D), jnp.float32),
                        pltpu.VMEM((_BR, 8), jnp.float32)],
    )(adj_mat, x_pad, f1m, f2t, b)
    return out


# speedup vs baseline: 3.3207x; 1.5655x over previous
import jax
import jax.numpy as jnp
from jax.experimental import pallas as pl
from jax.experimental.pallas import tpu as pltpu

_N = 10000
_D = 128
_BR = 400
_BC = 2048
_NRB = _N // _BR
_NCB = -(-_N // _BC)
_NP = _NCB * _BC
_BP = 2000


def _proj_kernel(in_ref, w_ref, wf_ref, x_ref, f_ref):
    x = jnp.dot(in_ref[...], w_ref[...], preferred_element_type=jnp.float32)
    x_ref[...] = x
    ff = jnp.dot(x, wf_ref[...], preferred_element_type=jnp.float32)
    f_ref[...] = jnp.where(ff > 0, ff, jnp.exp(ff) - 1.0)


def _gat_kernel(adj_ref, xv_ref, f1m_ref, f2t_ref, b_ref, out_ref, acc_ref):
    j = pl.program_id(1)

    @pl.when(j == 0)
    def _():
        acc_ref[...] = jnp.zeros_like(acc_ref)

    adj = adj_ref[...].astype(jnp.bfloat16)
    xv = xv_ref[...]
    zero = jnp.bfloat16(0.0)
    for h in range(2):
        srow = f1m_ref[:, h:h + 1].astype(jnp.bfloat16)
        m = f1m_ref[:, 2 + h:3 + h].astype(jnp.bfloat16)
        scol = f2t_ref[h:h + 1, :].astype(jnp.bfloat16)
        z = adj * (srow + scol)
        z = jnp.maximum(z, jnp.bfloat16(0.2) * z)
        zs = z - m
        e = jnp.where(zs <= zero, jnp.exp2(zs), zero)
        acc_ref[h, :, :] += jnp.dot(e, xv, preferred_element_type=jnp.float32)

    @pl.when(j == _NCB - 1)
    def _():
        res = None
        for h in range(2):
            num = acc_ref[h, :, 0:_D]
            den = acc_ref[h, :, _D:_D + 1]
            v = num / den + b_ref[h:h + 1, :]
            v = jnp.where(v > 0, v, jnp.exp(v) - 1.0)
            res = v if res is None else res + v
        out_ref[...] = res * 0.5


def kernel(inputs, adj_mat, W, w1, w2, b):
    wf = jnp.concatenate(
        [w1[0], w1[1], w2[0], w2[1], jnp.zeros((_D, 4), jnp.float32)], axis=1)
    x, f = pl.pallas_call(
        _proj_kernel,
        grid=(_N // _BP,),
        in_specs=[pl.BlockSpec((_BP, _D), lambda i: (i, 0)),
                  pl.BlockSpec((_D, _D), lambda i: (0, 0)),
                  pl.BlockSpec((_D, 8), lambda i: (0, 0))],
        out_specs=[pl.BlockSpec((_BP, _D), lambda i: (i, 0)),
                   pl.BlockSpec((_BP, 8), lambda i: (i, 0))],
        out_shape=(jax.ShapeDtypeStruct((_N, _D), jnp.float32),
                   jax.ShapeDtypeStruct((_N, 8), jnp.float32)),
    )(inputs, W, wf)

    log2e = jnp.float32(1.4426950408889634)
    f1 = f[:, 0:2] * log2e
    f2 = f[:, 2:4] * log2e
    m = jnp.maximum(f1 + jnp.max(f2, axis=0)[None, :], 0.0)
    m = m * 1.01 + jnp.float32(0.01)
    f1m = jnp.concatenate([f1, m], axis=1)
    f2t = jnp.pad(f2.T, ((0, 6), (0, _NP - _N)))
    xv = jnp.concatenate(
        [x, jnp.ones((_N, 1), jnp.float32), jnp.zeros((_N, 127), jnp.float32)],
        axis=1)
    xv = jnp.pad(xv, ((0, _NP - _N), (0, 0))).astype(jnp.bfloat16)

    out = pl.pallas_call(
        _gat_kernel,
        grid=(_NRB, _NCB),
        in_specs=[pl.BlockSpec((_BR, _BC), lambda i, j: (i, j)),
                  pl.BlockSpec((_BC, 256), lambda i, j: (j, 0)),
                  pl.BlockSpec((_BR, 4), lambda i, j: (i, 0)),
                  pl.BlockSpec((8, _BC), lambda i, j: (0, j)),
                  pl.BlockSpec((2, _D), lambda i, j: (0, 0))],
        out_specs=pl.BlockSpec((_BR, _D), lambda i, j: (i, 0)),
        out_shape=jax.ShapeDtypeStruct((_N, _D), jnp.float32),
        scratch_shapes=[pltpu.VMEM((2, _BR, 256), jnp.float32)],
    )(adj_mat, xv, f1m, f2t, b)
    return out


# speedup vs baseline: 4.6757x; 1.4081x over previous
import jax
import jax.numpy as jnp
from jax.experimental import pallas as pl
from jax.experimental.pallas import tpu as pltpu

_N = 10000
_D = 128
_BR = 1000
_BC = 2048
_NRB = _N // _BR
_NCB = -(-_N // _BC)
_BP = 2048
_NPB = -(-_N // _BP)


def _proj_kernel(in_ref, w_ref, wf_ref, xv_ref, f_ref):
    i = pl.program_id(0)
    x = jnp.dot(in_ref[...], w_ref[...], preferred_element_type=jnp.float32)
    ff = jnp.dot(x, wf_ref[...], preferred_element_type=jnp.float32)
    f_ref[...] = jnp.where(ff > 0, ff, jnp.exp(ff) - 1.0)
    row = jax.lax.broadcasted_iota(jnp.int32, (_BP, 256), 0)
    rv = row < (_N - i * _BP)
    xv = jnp.concatenate(
        [x, jnp.ones((_BP, 1), jnp.float32), jnp.zeros((_BP, 127), jnp.float32)],
        axis=1)
    xv_ref[...] = jnp.where(rv, xv, 0.0).astype(jnp.bfloat16)


def _gat_kernel(adj_ref, xv_ref, f1_ref, f2t_ref, b_ref, out_ref, acc_ref):
    j = pl.program_id(1)

    @pl.when(j == 0)
    def _():
        acc_ref[...] = jnp.zeros_like(acc_ref)

    def compute(tail):
        adj = adj_ref[...].astype(jnp.bfloat16)
        xv = xv_ref[...]
        for h in range(2):
            srow = f1_ref[:, h:h + 1].astype(jnp.bfloat16)
            scol = f2t_ref[h:h + 1, :].astype(jnp.bfloat16)
            t = srow + scol
            t = jnp.maximum(t, jnp.bfloat16(0.2) * t)
            e = jnp.exp2(adj * t)
            if tail:
                col = jax.lax.broadcasted_iota(jnp.int32, (_BR, _BC), 1)
                e = jnp.where(col < (_N - j * _BC), e, jnp.bfloat16(0.0))
            acc_ref[h, :, :] += jnp.dot(e, xv,
                                        preferred_element_type=jnp.float32)

    @pl.when(j < _NCB - 1)
    def _():
        compute(False)

    @pl.when(j == _NCB - 1)
    def _():
        compute(True)
        res = None
        for h in range(2):
            num = acc_ref[h, :, 0:_D]
            den = acc_ref[h, :, _D:_D + 1]
            v = num / den + b_ref[h:h + 1, :]
            v = jnp.where(v > 0, v, jnp.exp(v) - 1.0)
            res = v if res is None else res + v
        out_ref[...] = res * 0.5


def kernel(inputs, adj_mat, W, w1, w2, b):
    wf = jnp.concatenate(
        [w1[0], w1[1], w2[0], w2[1], jnp.zeros((_D, 4), jnp.float32)], axis=1)
    xv, f = pl.pallas_call(
        _proj_kernel,
        grid=(_NPB,),
        in_specs=[pl.BlockSpec((_BP, _D), lambda i: (i, 0)),
                  pl.BlockSpec((_D, _D), lambda i: (0, 0)),
                  pl.BlockSpec((_D, 8), lambda i: (0, 0))],
        out_specs=[pl.BlockSpec((_BP, 256), lambda i: (i, 0)),
                   pl.BlockSpec((_BP, 8), lambda i: (i, 0))],
        out_shape=(jax.ShapeDtypeStruct((_NPB * _BP, 256), jnp.bfloat16),
                   jax.ShapeDtypeStruct((_NPB * _BP, 8), jnp.float32)),
    )(inputs, W, wf)

    log2e = jnp.float32(1.4426950408889634)
    f1 = f[:_N, 0:2] * log2e
    f2t = jnp.pad(f[:_N, 2:4].T * log2e,
                  ((0, 6), (0, _NCB * _BC - _N)))

    out = pl.pallas_call(
        _gat_kernel,
        grid=(_NRB, _NCB),
        in_specs=[pl.BlockSpec((_BR, _BC), lambda i, j: (i, j)),
                  pl.BlockSpec((_BC, 256), lambda i, j: (j, 0)),
                  pl.BlockSpec((_BR, 2), lambda i, j: (i, 0)),
                  pl.BlockSpec((8, _BC), lambda i, j: (0, j)),
                  pl.BlockSpec((2, _D), lambda i, j: (0, 0))],
        out_specs=pl.BlockSpec((_BR, _D), lambda i, j: (i, 0)),
        out_shape=jax.ShapeDtypeStruct((_N, _D), jnp.float32),
        scratch_shapes=[pltpu.VMEM((2, _BR, 256), jnp.float32)],
    )(adj_mat, xv, f1, f2t, b)
    return out


# speedup vs baseline: 5.2546x; 1.1238x over previous
import jax
import jax.numpy as jnp
from jax.experimental import pallas as pl
from jax.experimental.pallas import tpu as pltpu

_N = 10000
_D = 128
_BR = 400
_NRB = _N // _BR
_BP = 2000


def _proj_kernel(in_ref, w_ref, wf_ref, xv_ref, f_ref):
    x = jnp.dot(in_ref[...], w_ref[...], preferred_element_type=jnp.float32)
    ff = jnp.dot(x, wf_ref[...], preferred_element_type=jnp.float32)
    f_ref[...] = jnp.where(ff > 0, ff, jnp.exp(ff) - 1.0)
    xv = jnp.concatenate(
        [x, jnp.ones((_BP, 1), jnp.float32), jnp.zeros((_BP, 127), jnp.float32)],
        axis=1)
    xv_ref[...] = xv.astype(jnp.bfloat16)


def _gat_kernel(adj_ref, xv_ref, f1_ref, f2t_ref, b_ref, out_ref):
    adj = adj_ref[...].astype(jnp.bfloat16)
    xv = xv_ref[...]
    res = None
    for h in range(2):
        srow = f1_ref[:, h:h + 1].astype(jnp.bfloat16)
        scol = f2t_ref[h:h + 1, :].astype(jnp.bfloat16)
        t = srow + scol
        t = jnp.maximum(t, jnp.bfloat16(0.2) * t)
        e = jnp.exp2(adj * t)
        acc = jnp.dot(e, xv, preferred_element_type=jnp.float32)
        v = acc[:, 0:_D] / acc[:, _D:_D + 1] + b_ref[h:h + 1, :]
        v = jnp.where(v > 0, v, jnp.exp(v) - 1.0)
        res = v if res is None else res + v
    out_ref[...] = res * 0.5


def kernel(inputs, adj_mat, W, w1, w2, b):
    wf = jnp.concatenate(
        [w1[0], w1[1], w2[0], w2[1], jnp.zeros((_D, 4), jnp.float32)], axis=1)
    xv, f = pl.pallas_call(
        _proj_kernel,
        grid=(_N // _BP,),
        in_specs=[pl.BlockSpec((_BP, _D), lambda i: (i, 0)),
                  pl.BlockSpec((_D, _D), lambda i: (0, 0)),
                  pl.BlockSpec((_D, 8), lambda i: (0, 0))],
        out_specs=[pl.BlockSpec((_BP, 256), lambda i: (i, 0)),
                   pl.BlockSpec((_BP, 8), lambda i: (i, 0))],
        out_shape=(jax.ShapeDtypeStruct((_N, 256), jnp.bfloat16),
                   jax.ShapeDtypeStruct((_N, 8), jnp.float32)),
    )(inputs, W, wf)

    log2e = jnp.float32(1.4426950408889634)
    f1 = f[:, 0:2] * log2e
    f2t = jnp.pad(f[:, 2:4].T * log2e, ((0, 6), (0, 0)))

    out = pl.pallas_call(
        _gat_kernel,
        grid=(_NRB,),
        in_specs=[pl.BlockSpec((_BR, _N), lambda i: (i, 0)),
                  pl.BlockSpec((_N, 256), lambda i: (0, 0)),
                  pl.BlockSpec((_BR, 2), lambda i: (i, 0)),
                  pl.BlockSpec((8, _N), lambda i: (0, 0)),
                  pl.BlockSpec((2, _D), lambda i: (0, 0))],
        out_specs=pl.BlockSpec((_BR, _D), lambda i: (i, 0)),
        out_shape=jax.ShapeDtypeStruct((_N, _D), jnp.float32),
    )(adj_mat, xv, f1, f2t, b)
    return out
